# baseline (speedup 1.0000x reference)
import functools
import jax
import jax.numpy as jnp
from jax import lax
from jax.experimental import pallas as pl
from jax.experimental.pallas import tpu as pltpu
from jax.experimental.pallas import tpu_sc as plsc

_NC = 2
_NS = 16
_NW = _NC * _NS
_L = 16

_N_LEVELS = 2.0 ** 8 - 1.0
_CH = 32768

_mesh = plsc.VectorSubcoreMesh(core_axis_name="c", subcore_axis_name="s")


def _make_absmax(n):
    w = n // _NW
    nch = w // _CH
    nvec = _CH // _L

    @functools.partial(
        pl.kernel,
        out_type=jax.ShapeDtypeStruct((_NW * _L,), jnp.float32),
        mesh=_mesh,
        scratch_types=[
            pltpu.VMEM((_CH,), jnp.float32),
            pltpu.VMEM((_CH,), jnp.float32),
            pltpu.VMEM((_L,), jnp.float32),
            pltpu.SemaphoreType.DMA,
            pltpu.SemaphoreType.DMA,
        ],
    )
    def absmax_kernel(x_hbm, out_hbm, b0, b1, mx, s0, s1):
        wid = lax.axis_index("s") * _NC + lax.axis_index("c")
        base = wid * w
        bufs = (b0, b1)
        sems = (s0, s1)
        cps = [None, None]
        cps[0] = pltpu.async_copy(x_hbm.at[pl.ds(base, _CH)], b0, s0)
        acc = jnp.zeros((_L,), jnp.float32)
        for j in range(nch):
            k = j % 2
            if j + 1 < nch:
                kn = (j + 1) % 2
                cps[kn] = pltpu.async_copy(
                    x_hbm.at[pl.ds(base + (j + 1) * _CH, _CH)], bufs[kn],
                    sems[kn])
            cps[k].wait()
            b = bufs[k]

            def _absmax_body(i, a, b=b):
                return jnp.maximum(a, jnp.abs(b[pl.ds(i * _L, _L)]))

            acc = plsc.parallel_loop(0, nvec, unroll=8,
                                     carry=acc)(_absmax_body)

        mx[...] = acc
        pltpu.sync_copy(mx, out_hbm.at[pl.ds(wid * _L, _L)])

    return absmax_kernel


def _make_quant(n):
    w = n // _NW
    nch = w // _CH
    nvec = _CH // _L

    @functools.partial(
        pl.kernel,
        out_type=jax.ShapeDtypeStruct((n,), jnp.float32),
        mesh=_mesh,
        scratch_types=[
            pltpu.VMEM((_CH,), jnp.float32),
            pltpu.VMEM((_CH,), jnp.float32),
            pltpu.VMEM((3 * _L,), jnp.float32),
            pltpu.SemaphoreType.DMA,
            pltpu.SemaphoreType.DMA,
            pltpu.SemaphoreType.DMA,
            pltpu.SemaphoreType.DMA,
        ],
    )
    def quant_kernel(x_hbm, p_hbm, out_hbm, b0, b1, pv, g0, g1, t0, t1):
        wid = lax.axis_index("s") * _NC + lax.axis_index("c")
        base = wid * w
        pltpu.sync_copy(p_hbm, pv)
        scale = pv[pl.ds(0, _L)]
        zp = pv[pl.ds(_L, _L)]
        inv = pv[pl.ds(2 * _L, _L)]
        bufs = (b0, b1)
        gsems = (g0, g1)
        tsems = (t0, t1)
        cpg = [None, None]
        cpt = [None, None]
        cpg[0] = pltpu.async_copy(x_hbm.at[pl.ds(base, _CH)], b0, g0)
        for j in range(nch):
            k = j % 2
            cpg[k].wait()
            if j + 1 < nch:
                kn = (j + 1) % 2
                if cpt[kn] is not None:
                    cpt[kn].wait()
                cpg[kn] = pltpu.async_copy(
                    x_hbm.at[pl.ds(base + (j + 1) * _CH, _CH)], bufs[kn],
                    gsems[kn])
            b = bufs[k]

            def _quant_body(i, b=b):
                x = b[pl.ds(i * _L, _L)]
                y = x * scale - zp
                q = (y + 12582912.0) - 12582912.0
                b[pl.ds(i * _L, _L)] = (q + zp) * inv

            plsc.parallel_loop(0, nvec, unroll=8)(_quant_body)

            cpt[k] = pltpu.async_copy(
                bufs[k], out_hbm.at[pl.ds(base + j * _CH, _CH)], tsems[k])
        cpt[0].wait()
        cpt[1].wait()

    return quant_kernel


def kernel(tensor, image_size):
    rows, cols = tensor.shape
    n = rows * cols
    flat = tensor.reshape(n)
    partials = _make_absmax(n)(flat)
    alpha = jnp.max(partials)
    d = jnp.maximum(2.0 * alpha, 1e-8)
    scale = _N_LEVELS / d
    zp = scale * (-alpha)
    inv = d / _N_LEVELS
    params = jnp.concatenate([
        jnp.full((_L,), scale, jnp.float32),
        jnp.full((_L,), zp, jnp.float32),
        jnp.full((_L,), inv, jnp.float32),
    ])
    out = _make_quant(n)(flat, params)
    return out.reshape(rows, cols)


# speedup vs baseline: 5.6907x; 5.6907x over previous
import jax
import jax.numpy as jnp
from jax.experimental import pallas as pl
from jax.experimental.pallas import tpu as pltpu

_N_LEVELS = 2.0 ** 8 - 1.0
_NA = 16
_NB = 8


def _body(in_hbm, out_hbm, buf, ostage, isem, osem):
    rows, cols = buf.shape
    ra = rows // _NA
    rb = rows // _NB

    def copy_in(i, k):
        return pltpu.make_async_copy(
            in_hbm.at[pl.ds(i * ra, ra), :],
            buf.at[pl.ds(i * ra, ra), :],
            isem.at[k])

    def copy_out(i, k):
        return pltpu.make_async_copy(
            ostage.at[pl.ds(pl.multiple_of(k * rb, rb), rb), :],
            out_hbm.at[pl.ds(i * rb, rb), :],
            osem.at[k])

    copy_in(0, 0).start()

    def phase_a(i, m):
        @pl.when(i + 1 < _NA)
        def _():
            copy_in(i + 1, (i + 1) % 2).start()

        copy_in(i, i % 2).wait()
        band = buf[pl.ds(i * ra, ra), :]
        return jnp.maximum(m, jnp.max(jnp.abs(band)))

    alpha = jax.lax.fori_loop(0, _NA, phase_a, jnp.float32(0.0))

    d = jnp.maximum(2.0 * alpha, 1e-8)
    scale = _N_LEVELS / d
    zp = scale * (-alpha)
    inv = d * (1.0 / _N_LEVELS)

    def phase_b(i, carry):
        k = i % 2

        @pl.when(i >= 2)
        def _():
            copy_out(i - 2, k).wait()

        x = buf[pl.ds(i * rb, rb), :]
        q = jnp.round(x * scale - zp)
        ostage[pl.ds(pl.multiple_of(k * rb, rb), rb), :] = (q + zp) * inv
        copy_out(i, k).start()
        return carry

    jax.lax.fori_loop(0, _NB, phase_b, 0)
    copy_out(_NB - 2, (_NB - 2) % 2).wait()
    copy_out(_NB - 1, (_NB - 1) % 2).wait()


def kernel(tensor, image_size):
    rows, cols = tensor.shape
    rb = rows // _NB
    return pl.pallas_call(
        _body,
        in_specs=[pl.BlockSpec(memory_space=pl.ANY)],
        out_specs=pl.BlockSpec(memory_space=pl.ANY),
        out_shape=jax.ShapeDtypeStruct((rows, cols), tensor.dtype),
        scratch_shapes=[
            pltpu.VMEM((rows, cols), jnp.float32),
            pltpu.VMEM((2 * rb, cols), jnp.float32),
            pltpu.SemaphoreType.DMA((2,)),
            pltpu.SemaphoreType.DMA((2,)),
        ],
    )(tensor)


# speedup vs baseline: 8.8515x; 1.5554x over previous
import jax
import jax.numpy as jnp
from jax.experimental import pallas as pl
from jax.experimental.pallas import tpu as pltpu

_N_LEVELS = 2.0 ** 8 - 1.0
_NA = 8
_NB = 4


def _body(in_hbm, out_hbm, buf, ostage, isem, osem):
    rows, cols = buf.shape
    ra = rows // _NA
    rb = rows // _NB

    def copy_in(i, k):
        return pltpu.make_async_copy(
            in_hbm.at[pl.ds(i * ra, ra), :],
            buf.at[pl.ds(i * ra, ra), :],
            isem.at[k])

    def copy_out(i, k):
        return pltpu.make_async_copy(
            ostage.at[pl.ds(pl.multiple_of(k * rb, rb), rb), :],
            out_hbm.at[pl.ds(i * rb, rb), :],
            osem.at[k])

    for j in range(_NA):
        copy_in(j, j).start()

    def phase_a(i, m):
        copy_in(i, i).wait()
        band = buf[pl.ds(i * ra, ra), :]
        return jnp.maximum(m, jnp.max(jnp.abs(band)))

    alpha = jax.lax.fori_loop(0, _NA, phase_a, jnp.float32(0.0))

    d = jnp.maximum(2.0 * alpha, 1e-8)
    scale = _N_LEVELS / d
    zp = scale * (-alpha)
    inv = d * (1.0 / _N_LEVELS)

    def phase_b(i, carry):
        k = i % 2

        @pl.when(i >= 2)
        def _():
            copy_out(i - 2, k).wait()

        x = buf[pl.ds(i * rb, rb), :]
        q = jnp.round(x * scale - zp)
        ostage[pl.ds(pl.multiple_of(k * rb, rb), rb), :] = (q + zp) * inv
        copy_out(i, k).start()
        return carry

    jax.lax.fori_loop(0, _NB, phase_b, 0)
    copy_out(_NB - 2, (_NB - 2) % 2).wait()
    copy_out(_NB - 1, (_NB - 1) % 2).wait()


def kernel(tensor, image_size):
    rows, cols = tensor.shape
    rb = rows // _NB
    return pl.pallas_call(
        _body,
        in_specs=[pl.BlockSpec(memory_space=pl.ANY)],
        out_specs=pl.BlockSpec(memory_space=pl.ANY),
        out_shape=jax.ShapeDtypeStruct((rows, cols), tensor.dtype),
        scratch_shapes=[
            pltpu.VMEM((rows, cols), jnp.float32),
            pltpu.VMEM((2 * rb, cols), jnp.float32),
            pltpu.SemaphoreType.DMA((_NA,)),
            pltpu.SemaphoreType.DMA((2,)),
        ],
    )(tensor)
